# baseline (speedup 1.0000x reference)
import functools

import jax
import jax.numpy as jnp
from jax import lax
from jax.experimental import pallas as pl
from jax.experimental.pallas import tpu as pltpu
from jax.experimental.pallas import tpu_sc as plsc

B = 1024
S = 520
D_SEQ = 32
E_TIME = 32
E_VAR = 32
E_STRUC = 16
D_OUT = D_SEQ + E_VAR + E_TIME + E_STRUC
INPUT_DIM = 20

_NC = 2
_NS = 16
_NW = _NC * _NS
_ROWS_PER_W = B // _NW
_GCH = 104


def _time_pattern_body(t_ref, w_ref, b_ref, out_ref):
    rows = lax.broadcasted_iota(jnp.int32, (S, INPUT_DIM), 0)
    cols = lax.broadcasted_iota(jnp.int32, (S, INPUT_DIM), 1)
    onehot = (rows % INPUT_DIM == cols).astype(jnp.float32)
    m_t = onehot * t_ref[...]
    affine = (jnp.dot(m_t, w_ref[...], preferred_element_type=jnp.float32)
              + jnp.dot(onehot, b_ref[...], preferred_element_type=jnp.float32))
    ch = lax.broadcasted_iota(jnp.int32, (S, E_TIME), 1)
    out_ref[...] = jnp.where(ch == 0, affine, jnp.sin(affine))


def _time_pattern_tc(t_f32, embed_weight, embed_bias):
    return pl.pallas_call(
        _time_pattern_body,
        out_shape=jax.ShapeDtypeStruct((S, E_TIME), jnp.float32),
    )(t_f32, embed_weight, embed_bias)


def _sc_assemble(seq2d, vidx, sidx, pat520, var_table, struc_table):
    mesh = plsc.VectorSubcoreMesh(
        core_axis_name="c", subcore_axis_name="s",
        num_cores=_NC, num_subcores=_NS)

    @functools.partial(
        pl.kernel,
        out_type=jax.ShapeDtypeStruct((B * S, D_OUT), jnp.float32),
        mesh=mesh,
        compiler_params=pltpu.CompilerParams(use_tc_tiling_on_sc=False),
        scratch_types=[
            pltpu.VMEM((S, E_TIME), jnp.float32),
            pltpu.VMEM((2, S, D_SEQ), jnp.float32),
            pltpu.VMEM((2, S, E_VAR), jnp.float32),
            pltpu.VMEM((2, S, E_STRUC), jnp.float32),
            pltpu.VMEM((2, S), jnp.int32),
            pltpu.VMEM((2, S), jnp.int32),
            pltpu.SemaphoreType.DMA,
            pltpu.SemaphoreType.DMA,
            pltpu.SemaphoreType.DMA,
            pltpu.SemaphoreType.DMA,
            pltpu.SemaphoreType.DMA,
        ],
    )
    def k(seq_hbm, vi_hbm, si_hbm, pat_hbm, var_hbm, st_hbm, out_hbm,
          pat_v, seq_v, var_v, st_v, vi_v, si_v,
          sem_i0, sem_i1, sem_g, sem_w0, sem_w1):
        sem_i = (sem_i0, sem_i1)
        sem_w = (sem_w0, sem_w1)
        wid = lax.axis_index("s") * _NC + lax.axis_index("c")
        row0 = wid * _ROWS_PER_W

        pltpu.sync_copy(pat_hbm, pat_v)

        def fill_idx(r, b):
            base = (row0 + r) * S
            pltpu.async_copy(vi_hbm.at[pl.ds(base, S)], vi_v.at[b], sem_i[b])
            pltpu.async_copy(si_hbm.at[pl.ds(base, S)], si_v.at[b], sem_i[b])

        def wait_idx(b):
            pltpu.make_async_copy(
                vi_hbm.at[pl.ds(0, S)], vi_v.at[b], sem_i[b]).wait()
            pltpu.make_async_copy(
                si_hbm.at[pl.ds(0, S)], si_v.at[b], sem_i[b]).wait()

        def wait_writes(b):
            pltpu.make_async_copy(
                seq_v.at[b], out_hbm.at[pl.ds(0, S), 0:32], sem_w[b]).wait()
            pltpu.make_async_copy(
                var_v.at[b], out_hbm.at[pl.ds(0, S), 32:64], sem_w[b]).wait()
            pltpu.make_async_copy(
                pat_v, out_hbm.at[pl.ds(0, S), 64:96], sem_w[b]).wait()
            pltpu.make_async_copy(
                st_v.at[b], out_hbm.at[pl.ds(0, S), 96:112], sem_w[b]).wait()

        def do_row(r, b, first):
            wait_idx(b)
            if not first:
                wait_writes(b)
            base = (row0 + r) * S
            descs = [pltpu.async_copy(
                seq_hbm.at[pl.ds(base, S), :], seq_v.at[b], sem_g)]
            descs.append(pltpu.async_copy(
                var_hbm.at[vi_v.at[b]], var_v.at[b], sem_g))
            descs.append(pltpu.async_copy(
                st_hbm.at[si_v.at[b]], st_v.at[b], sem_g))
            for d in descs:
                d.wait()
            pltpu.async_copy(
                seq_v.at[b], out_hbm.at[pl.ds(base, S), 0:32], sem_w[b])
            pltpu.async_copy(
                var_v.at[b], out_hbm.at[pl.ds(base, S), 32:64], sem_w[b])
            pltpu.async_copy(
                pat_v, out_hbm.at[pl.ds(base, S), 64:96], sem_w[b])
            pltpu.async_copy(
                st_v.at[b], out_hbm.at[pl.ds(base, S), 96:112], sem_w[b])

        fill_idx(0, 0)
        fill_idx(1, 1)
        do_row(0, 0, True)
        fill_idx(2, 0)
        do_row(1, 1, True)
        fill_idx(3, 1)

        def body(i, carry):
            for b in (0, 1):
                r = 2 * i + b
                do_row(r, b, False)
                fill_idx(r + 2, b)
            return carry

        lax.fori_loop(1, _ROWS_PER_W // 2 - 1, body, 0)

        do_row(_ROWS_PER_W - 2, 0, False)
        do_row(_ROWS_PER_W - 1, 1, False)
        wait_writes(0)
        wait_writes(1)

    return k(seq2d, vidx, sidx, pat520, var_table, struc_table)


def kernel(sequence, time_index_sequence, variable_index_sequence,
           sector_index_sequence, embed_weight, embed_bias, var_table,
           struc_table):
    b, s, ds = sequence.shape
    t = time_index_sequence[0:1, :INPUT_DIM].astype(jnp.float32)
    pat520 = _time_pattern_tc(t, embed_weight, embed_bias)
    seq2d = sequence.reshape(b * s, ds)
    vi = variable_index_sequence.reshape(-1).astype(jnp.int32)
    si = sector_index_sequence.reshape(-1).astype(jnp.int32)
    out2d = _sc_assemble(seq2d, vi, si, pat520, var_table, struc_table)
    return out2d.reshape(b, s, D_OUT)


# speedup vs baseline: 1.8988x; 1.8988x over previous
import functools

import jax
import jax.numpy as jnp
from jax import lax
from jax.experimental import pallas as pl
from jax.experimental.pallas import tpu as pltpu
from jax.experimental.pallas import tpu_sc as plsc

B = 1024
S = 520
D_SEQ = 32
E_TIME = 32
E_VAR = 32
E_STRUC = 16
D_OUT = D_SEQ + E_VAR + E_TIME + E_STRUC
INPUT_DIM = 20

_NC = 2
_NS = 16
_NW = _NC * _NS
_ROWS_PER_W = B // _NW
_GCH = 104


def _time_pattern_body(t_ref, w_ref, b_ref, out_ref):
    rows = lax.broadcasted_iota(jnp.int32, (S, INPUT_DIM), 0)
    cols = lax.broadcasted_iota(jnp.int32, (S, INPUT_DIM), 1)
    onehot = (rows % INPUT_DIM == cols).astype(jnp.float32)
    m_t = onehot * t_ref[...]
    affine = (jnp.dot(m_t, w_ref[...], preferred_element_type=jnp.float32)
              + jnp.dot(onehot, b_ref[...], preferred_element_type=jnp.float32))
    ch = lax.broadcasted_iota(jnp.int32, (S, E_TIME), 1)
    out_ref[...] = jnp.where(ch == 0, affine, jnp.sin(affine))


def _time_pattern_tc(t_f32, embed_weight, embed_bias):
    return pl.pallas_call(
        _time_pattern_body,
        out_shape=jax.ShapeDtypeStruct((S, E_TIME), jnp.float32),
    )(t_f32, embed_weight, embed_bias)


def _sc_assemble(seq2d, vidx, sidx, pat520, var_table, struc_table):
    mesh = plsc.VectorSubcoreMesh(
        core_axis_name="c", subcore_axis_name="s",
        num_cores=_NC, num_subcores=_NS)

    @functools.partial(
        pl.kernel,
        out_type=jax.ShapeDtypeStruct((B * S, D_OUT), jnp.float32),
        mesh=mesh,
        compiler_params=pltpu.CompilerParams(use_tc_tiling_on_sc=False),
        scratch_types=[
            pltpu.VMEM((S, E_TIME), jnp.float32),
            pltpu.VMEM((2, S, D_SEQ), jnp.float32),
            pltpu.VMEM((2, S, E_VAR), jnp.float32),
            pltpu.VMEM((2, S, E_STRUC), jnp.float32),
            pltpu.VMEM((2, S), jnp.int32),
            pltpu.VMEM((2, S), jnp.int32),
            pltpu.SemaphoreType.DMA,
            pltpu.SemaphoreType.DMA,
            pltpu.SemaphoreType.DMA,
            pltpu.SemaphoreType.DMA,
            pltpu.SemaphoreType.DMA,
        ],
    )
    def k(seq_hbm, vi_hbm, si_hbm, pat_hbm, var_hbm, st_hbm, out_hbm,
          pat_v, seq_v, var_v, st_v, vi_v, si_v,
          sem_i0, sem_i1, sem_g, sem_w0, sem_w1):
        sem_i = (sem_i0, sem_i1)
        sem_w = (sem_w0, sem_w1)
        wid = lax.axis_index("s") * _NC + lax.axis_index("c")
        row0 = wid * _ROWS_PER_W

        pltpu.sync_copy(pat_hbm, pat_v)

        def fill_idx(r, b):
            base = (row0 + r) * S
            pltpu.async_copy(vi_hbm.at[pl.ds(base, S)], vi_v.at[b], sem_i[b])
            pltpu.async_copy(si_hbm.at[pl.ds(base, S)], si_v.at[b], sem_i[b])

        def wait_idx(b):
            pltpu.make_async_copy(
                vi_hbm.at[pl.ds(0, S)], vi_v.at[b], sem_i[b]).wait()
            pltpu.make_async_copy(
                si_hbm.at[pl.ds(0, S)], si_v.at[b], sem_i[b]).wait()

        def wait_writes(b):
            pltpu.make_async_copy(
                seq_v.at[b], out_hbm.at[pl.ds(0, S), 0:32], sem_w[b]).wait()
            pltpu.make_async_copy(
                var_v.at[b], out_hbm.at[pl.ds(0, S), 32:64], sem_w[b]).wait()
            pltpu.make_async_copy(
                pat_v, out_hbm.at[pl.ds(0, S), 64:96], sem_w[b]).wait()
            pltpu.make_async_copy(
                st_v.at[b], out_hbm.at[pl.ds(0, S), 96:112], sem_w[b]).wait()

        def do_row(r, b, first):
            wait_idx(b)
            if not first:
                wait_writes(b)
            base = (row0 + r) * S
            descs = [pltpu.async_copy(
                seq_hbm.at[pl.ds(base, S), :], seq_v.at[b], sem_g)]
            descs.append(pltpu.async_copy(
                var_hbm.at[pl.ds(wid * 26, 26)].at[vi_v.at[b]],
                var_v.at[b], sem_g))
            descs.append(pltpu.async_copy(
                st_hbm.at[pl.ds(wid * 26, 26)].at[si_v.at[b]],
                st_v.at[b], sem_g))
            for d in descs:
                d.wait()
            pltpu.async_copy(
                seq_v.at[b], out_hbm.at[pl.ds(base, S), 0:32], sem_w[b])
            pltpu.async_copy(
                var_v.at[b], out_hbm.at[pl.ds(base, S), 32:64], sem_w[b])
            pltpu.async_copy(
                pat_v, out_hbm.at[pl.ds(base, S), 64:96], sem_w[b])
            pltpu.async_copy(
                st_v.at[b], out_hbm.at[pl.ds(base, S), 96:112], sem_w[b])

        fill_idx(0, 0)
        fill_idx(1, 1)
        do_row(0, 0, True)
        fill_idx(2, 0)
        do_row(1, 1, True)
        fill_idx(3, 1)

        def body(i, carry):
            for b in (0, 1):
                r = 2 * i + b
                do_row(r, b, False)
                fill_idx(r + 2, b)
            return carry

        lax.fori_loop(1, _ROWS_PER_W // 2 - 1, body, 0)

        do_row(_ROWS_PER_W - 2, 0, False)
        do_row(_ROWS_PER_W - 1, 1, False)
        wait_writes(0)
        wait_writes(1)

    return k(seq2d, vidx, sidx, pat520, var_table, struc_table)


def kernel(sequence, time_index_sequence, variable_index_sequence,
           sector_index_sequence, embed_weight, embed_bias, var_table,
           struc_table):
    b, s, ds = sequence.shape
    t = time_index_sequence[0:1, :INPUT_DIM].astype(jnp.float32)
    pat520 = _time_pattern_tc(t, embed_weight, embed_bias)
    seq2d = sequence.reshape(b * s, ds)
    vi = variable_index_sequence.reshape(-1).astype(jnp.int32)
    si = sector_index_sequence.reshape(-1).astype(jnp.int32)
    var_rep = jnp.tile(var_table, (_NW, 1))
    st_rep = jnp.tile(struc_table, (_NW, 1))
    out2d = _sc_assemble(seq2d, vi, si, pat520, var_rep, st_rep)
    return out2d.reshape(b, s, D_OUT)


# speedup vs baseline: 2.1025x; 1.1072x over previous
import functools

import jax
import jax.numpy as jnp
from jax import lax
from jax.experimental import pallas as pl
from jax.experimental.pallas import tpu as pltpu
from jax.experimental.pallas import tpu_sc as plsc

B = 1024
S = 520
D_SEQ = 32
E_TIME = 32
E_VAR = 32
E_STRUC = 16
D_OUT = D_SEQ + E_VAR + E_TIME + E_STRUC
INPUT_DIM = 20

_NC = 2
_NS = 16
_NW = _NC * _NS
_ROWS_PER_W = B // _NW
_GCH = 104


def _time_pattern_body(t_ref, w_ref, b_ref, out_ref):
    rows = lax.broadcasted_iota(jnp.int32, (S, INPUT_DIM), 0)
    cols = lax.broadcasted_iota(jnp.int32, (S, INPUT_DIM), 1)
    onehot = (rows % INPUT_DIM == cols).astype(jnp.float32)
    m_t = onehot * t_ref[...]
    affine = (jnp.dot(m_t, w_ref[...], preferred_element_type=jnp.float32)
              + jnp.dot(onehot, b_ref[...], preferred_element_type=jnp.float32))
    ch = lax.broadcasted_iota(jnp.int32, (S, E_TIME), 1)
    out_ref[...] = jnp.where(ch == 0, affine, jnp.sin(affine))


def _time_pattern_tc(t_f32, embed_weight, embed_bias):
    return pl.pallas_call(
        _time_pattern_body,
        out_shape=jax.ShapeDtypeStruct((S, E_TIME), jnp.float32),
    )(t_f32, embed_weight, embed_bias)


_RB = 8


def _tc_pack_body(in_ref, out_ref):
    out_ref[...] = in_ref[:, :, :D_OUT]


def _tc_pack(xpad3):
    return pl.pallas_call(
        _tc_pack_body,
        grid=(B // _RB,),
        in_specs=[pl.BlockSpec((_RB, S, 128), lambda i: (i, 0, 0))],
        out_specs=pl.BlockSpec((_RB, S, D_OUT), lambda i: (i, 0, 0)),
        out_shape=jax.ShapeDtypeStruct((B, S, D_OUT), jnp.float32),
    )(xpad3)


def _sc_assemble(seq2d, vidx, sidx, pat520, var_table, struc_table):
    mesh = plsc.VectorSubcoreMesh(
        core_axis_name="c", subcore_axis_name="s",
        num_cores=_NC, num_subcores=_NS)

    @functools.partial(
        pl.kernel,
        out_type=jax.ShapeDtypeStruct((B * S, 128), jnp.float32),
        mesh=mesh,
        compiler_params=pltpu.CompilerParams(use_tc_tiling_on_sc=False),
        scratch_types=[
            pltpu.VMEM((S, E_TIME), jnp.float32),
            pltpu.VMEM((2, S, D_SEQ), jnp.float32),
            pltpu.VMEM((2, S, E_VAR), jnp.float32),
            pltpu.VMEM((2, S, E_STRUC), jnp.float32),
            pltpu.VMEM((2, S), jnp.int32),
            pltpu.VMEM((2, S), jnp.int32),
            pltpu.SemaphoreType.DMA,
            pltpu.SemaphoreType.DMA,
            pltpu.SemaphoreType.DMA,
            pltpu.SemaphoreType.DMA,
            pltpu.SemaphoreType.DMA,
        ],
    )
    def k(seq_hbm, vi_hbm, si_hbm, pat_hbm, var_hbm, st_hbm, out_hbm,
          pat_v, seq_v, var_v, st_v, vi_v, si_v,
          sem_i0, sem_i1, sem_g, sem_w0, sem_w1):
        sem_i = (sem_i0, sem_i1)
        sem_w = (sem_w0, sem_w1)
        wid = lax.axis_index("s") * _NC + lax.axis_index("c")
        row0 = wid * _ROWS_PER_W

        pltpu.sync_copy(pat_hbm, pat_v)

        def fill_idx(r, b):
            base = (row0 + r) * S
            pltpu.async_copy(vi_hbm.at[pl.ds(base, S)], vi_v.at[b], sem_i[b])
            pltpu.async_copy(si_hbm.at[pl.ds(base, S)], si_v.at[b], sem_i[b])

        def wait_idx(b):
            pltpu.make_async_copy(
                vi_hbm.at[pl.ds(0, S)], vi_v.at[b], sem_i[b]).wait()
            pltpu.make_async_copy(
                si_hbm.at[pl.ds(0, S)], si_v.at[b], sem_i[b]).wait()

        def wait_writes(b):
            pltpu.make_async_copy(
                seq_v.at[b], out_hbm.at[pl.ds(0, S), 0:32], sem_w[b]).wait()
            pltpu.make_async_copy(
                var_v.at[b], out_hbm.at[pl.ds(0, S), 32:64], sem_w[b]).wait()
            pltpu.make_async_copy(
                pat_v, out_hbm.at[pl.ds(0, S), 64:96], sem_w[b]).wait()
            pltpu.make_async_copy(
                st_v.at[b], out_hbm.at[pl.ds(0, S), 96:112], sem_w[b]).wait()

        def do_row(r, b, first):
            wait_idx(b)
            if not first:
                wait_writes(b)
            base = (row0 + r) * S
            descs = [pltpu.async_copy(
                seq_hbm.at[pl.ds(base, S), :], seq_v.at[b], sem_g)]
            descs.append(pltpu.async_copy(
                var_hbm.at[pl.ds(wid * 26, 26)].at[vi_v.at[b]],
                var_v.at[b], sem_g))
            descs.append(pltpu.async_copy(
                st_hbm.at[pl.ds(wid * 26, 26)].at[si_v.at[b]],
                st_v.at[b], sem_g))
            for d in descs:
                d.wait()
            pltpu.async_copy(
                seq_v.at[b], out_hbm.at[pl.ds(base, S), 0:32], sem_w[b])
            pltpu.async_copy(
                var_v.at[b], out_hbm.at[pl.ds(base, S), 32:64], sem_w[b])
            pltpu.async_copy(
                pat_v, out_hbm.at[pl.ds(base, S), 64:96], sem_w[b])
            pltpu.async_copy(
                st_v.at[b], out_hbm.at[pl.ds(base, S), 96:112], sem_w[b])

        fill_idx(0, 0)
        fill_idx(1, 1)
        do_row(0, 0, True)
        fill_idx(2, 0)
        do_row(1, 1, True)
        fill_idx(3, 1)

        def body(i, carry):
            for b in (0, 1):
                r = 2 * i + b
                do_row(r, b, False)
                fill_idx(r + 2, b)
            return carry

        lax.fori_loop(1, _ROWS_PER_W // 2 - 1, body, 0)

        do_row(_ROWS_PER_W - 2, 0, False)
        do_row(_ROWS_PER_W - 1, 1, False)
        wait_writes(0)
        wait_writes(1)

    return k(seq2d, vidx, sidx, pat520, var_table, struc_table)


def kernel(sequence, time_index_sequence, variable_index_sequence,
           sector_index_sequence, embed_weight, embed_bias, var_table,
           struc_table):
    b, s, ds = sequence.shape
    t = time_index_sequence[0:1, :INPUT_DIM].astype(jnp.float32)
    pat520 = _time_pattern_tc(t, embed_weight, embed_bias)
    seq2d = sequence.reshape(b * s, ds)
    vi = variable_index_sequence.reshape(-1).astype(jnp.int32)
    si = sector_index_sequence.reshape(-1).astype(jnp.int32)
    var_rep = jnp.tile(var_table, (_NW, 1))
    st_rep = jnp.tile(struc_table, (_NW, 1))
    out_pad = _sc_assemble(seq2d, vi, si, pat520, var_rep, st_rep)
    return _tc_pack(out_pad.reshape(b, s, 128))
